# Initial kernel scaffold; baseline (speedup 1.0000x reference)
import jax
import jax.numpy as jnp
from jax.experimental import pallas as pl


def kernel(x, edge_index, edge_attr, batch, fingerprint, lin_W_0, lin_b_0, upd_W_0, upd_b_0, lin_W_1, lin_b_1, upd_W_1, upd_b_1, lin_W_2, lin_b_2, upd_W_2, upd_b_2, lin_W_3, lin_b_3, upd_W_3, upd_b_3, gate_W1, gate_b1, gate_W2, gate_b2, fc1_W, fc1_b, bn_gamma, bn_beta, fc2_W, fc2_b):
    raise NotImplementedError("write your pallas kernel here")


# speedup vs baseline: 2.6095x; 2.6095x over previous
import functools

import jax
import jax.numpy as jnp
from jax import lax
from jax.experimental import pallas as pl
from jax.experimental.pallas import tpu as pltpu
from jax.experimental.pallas import tpu_sc as plsc

_N = 10000
_E = 320000
_H = 128
_G = 64
_NLAYER = 4

_NC = 2
_NS = 16
_LANES = 16
_NW = _NC * _NS
_CH = 128
_EPW = -(-_E // (_NW * _CH)) * _CH
_EPAD = _EPW * _NW
_NTBL = 10240
_DUMMY = _N


def _dot_t(a, w):
    return lax.dot_general(a, w, (((1,), (1,)), ((), ())),
                           precision=lax.Precision.HIGHEST,
                           preferred_element_type=jnp.float32)


def _sc_edge_pass(p, src, dst, ea, we):
    mesh = plsc.VectorSubcoreMesh(core_axis_name="c", subcore_axis_name="s")

    @functools.partial(
        pl.kernel,
        mesh=mesh,
        out_type=jax.ShapeDtypeStruct((_NC, _NTBL, _H), jnp.float32),
        scratch_types=[
            pltpu.VMEM((_CH,), jnp.int32),
            pltpu.VMEM((_CH,), jnp.int32),
            pltpu.VMEM((_CH,), jnp.float32),
            pltpu.VMEM((_CH, _H), jnp.float32),
            pltpu.VMEM((_H,), jnp.float32),
            pltpu.VMEM_SHARED((_NTBL, _H), jnp.float32),
            pltpu.SemaphoreType.DMA,
        ],
    )
    def k(p_hbm, src_hbm, dst_hbm, ea_hbm, we_hbm, out_hbm,
          src_v, dst_v, ea_v, rows_v, we_v, aggr_sh, sem):
        c = lax.axis_index("c")
        s = lax.axis_index("s")

        def zrow(r, carry):
            for t in range(_H // _LANES):
                rows_v[r, pl.ds(t * _LANES, _LANES)] = jnp.zeros(
                    (_LANES,), jnp.float32)
            return carry
        lax.fori_loop(0, _CH, zrow, 0)
        zpt = _NTBL // _NS
        for b in range(zpt // _CH):
            pltpu.sync_copy(rows_v, aggr_sh.at[pl.ds(s * zpt + b * _CH, _CH)])
        pltpu.sync_copy(we_hbm, we_v)
        plsc.subcore_barrier()

        base = (c * _NS + s) * _EPW

        def chunk_body(i, carry):
            off = pl.multiple_of(base + i * _CH, _CH)
            pltpu.sync_copy(src_hbm.at[pl.ds(off, _CH)], src_v)
            pltpu.sync_copy(dst_hbm.at[pl.ds(off, _CH)], dst_v)
            pltpu.sync_copy(ea_hbm.at[pl.ds(off, _CH)], ea_v)
            pltpu.async_copy(p_hbm.at[src_v], rows_v, sem).wait()

            def grp_body(g, gcarry):
                eav = ea_v[pl.ds(g * _LANES, _LANES)]
                for j in range(_LANES):
                    e = g * _LANES + j
                    eab = eav[j]
                    for t in range(_H // _LANES):
                        sl = rows_v[e, pl.ds(t * _LANES, _LANES)]
                        w = we_v[pl.ds(t * _LANES, _LANES)]
                        rows_v[e, pl.ds(t * _LANES, _LANES)] = jnp.maximum(
                            sl + eab * w, 0.0)
                return gcarry
            lax.fori_loop(0, _CH // _LANES, grp_body, 0)

            pltpu.sync_copy(rows_v, aggr_sh.at[dst_v], add=True)
            return carry
        lax.fori_loop(0, _EPW // _CH, chunk_body, 0)

        plsc.subcore_barrier()
        rpt = _NTBL // _NS
        pltpu.sync_copy(aggr_sh.at[pl.ds(s * rpt, rpt)],
                        out_hbm.at[c, pl.ds(s * rpt, rpt)])

    return k(p, src, dst, ea, we)


def _p0_body(x_ref, w_ref, b_ref, o_ref):
    o_ref[...] = _dot_t(x_ref[...], w_ref[...]) + b_ref[...]


def _mid_body(a_ref, p_ref, uw_ref, ub_ref, nw_ref, nb_ref, pn_ref):
    a = a_ref[0, :_N] + a_ref[1, :_N] + jnp.maximum(p_ref[...], 0.0)
    h = jnp.maximum(_dot_t(a, uw_ref[...]) + ub_ref[...], 0.0)
    pn_ref[...] = _dot_t(h, nw_ref[...]) + nb_ref[...]


def _tail_body(a_ref, p_ref, uw_ref, ub_ref, batch_ref, fp_ref,
               gw1_ref, gb1_ref, gw2_ref, gb2_ref,
               f1a_ref, f1b_ref, f1b_bias_ref, gamma_ref, beta_ref,
               f2w_ref, f2b_ref, o_ref):
    a = a_ref[0, :_N] + a_ref[1, :_N] + jnp.maximum(p_ref[...], 0.0)
    h = jnp.maximum(_dot_t(a, uw_ref[...]) + ub_ref[...], 0.0)
    g1 = jnp.tanh(_dot_t(h, gw1_ref[...]) + gb1_ref[...])
    gate = jax.nn.sigmoid(
        lax.dot_general(gw2_ref[...], g1, (((1,), (1,)), ((), ())),
                        precision=lax.Precision.HIGHEST,
                        preferred_element_type=jnp.float32)
        + gb2_ref[0, 0])
    seg = lax.broadcasted_iota(jnp.int32, (_G, _N), 0)
    m = (batch_ref[...] == seg).astype(jnp.float32) * gate
    pooled = jnp.dot(m, h, precision=lax.Precision.HIGHEST,
                     preferred_element_type=jnp.float32)
    z = (_dot_t(pooled, f1a_ref[...]) + _dot_t(fp_ref[...], f1b_ref[...])
         + f1b_bias_ref[...])
    z = z * gamma_ref[...] + beta_ref[...]
    z = jnp.maximum(z, 0.0)
    o_ref[...] = lax.dot_general(
        f2w_ref[...], z, (((1,), (1,)), ((), ())),
        precision=lax.Precision.HIGHEST,
        preferred_element_type=jnp.float32) + f2b_ref[0, 0]


def _tc_call(body, out_shape, *args):
    return pl.pallas_call(body, out_shape=out_shape)(*args)


def kernel(x, edge_index, edge_attr, batch, fingerprint,
           lin_W_0, lin_b_0, upd_W_0, upd_b_0,
           lin_W_1, lin_b_1, upd_W_1, upd_b_1,
           lin_W_2, lin_b_2, upd_W_2, upd_b_2,
           lin_W_3, lin_b_3, upd_W_3, upd_b_3,
           gate_W1, gate_b1, gate_W2, gate_b2,
           fc1_W, fc1_b, bn_gamma, bn_beta, fc2_W, fc2_b):
    lin_W = [lin_W_0, lin_W_1, lin_W_2, lin_W_3]
    lin_b = [lin_b_0, lin_b_1, lin_b_2, lin_b_3]
    upd_W = [upd_W_0, upd_W_1, upd_W_2, upd_W_3]
    upd_b = [upd_b_0, upd_b_1, upd_b_2, upd_b_3]

    wh = [w[:, :_H] for w in lin_W]
    we = [w[:, _H] for w in lin_W]
    lb = [b.reshape(1, _H) for b in lin_b]
    ub = [b.reshape(1, _H) for b in upd_b]

    pad = _EPAD - _E
    src_p = jnp.concatenate([edge_index[0],
                             jnp.zeros((pad,), jnp.int32)])
    dst_p = jnp.concatenate([edge_index[1],
                             jnp.full((pad,), _DUMMY, jnp.int32)])
    ea_p = jnp.concatenate([edge_attr[:, 0],
                            jnp.zeros((pad,), jnp.float32)])

    f32 = jnp.float32
    p = _tc_call(_p0_body, jax.ShapeDtypeStruct((_N, _H), f32),
                 x, wh[0], lb[0])

    for l in range(_NLAYER - 1):
        aggr = _sc_edge_pass(p, src_p, dst_p, ea_p, we[l])
        p = _tc_call(_mid_body, jax.ShapeDtypeStruct((_N, _H), f32),
                     aggr, p, upd_W[l], ub[l], wh[l + 1], lb[l + 1])

    aggr = _sc_edge_pass(p, src_p, dst_p, ea_p, we[_NLAYER - 1])

    fpk = fingerprint.shape[1]
    kp = -(-fpk // _H) * _H
    fp_pad = jnp.pad(fingerprint, ((0, 0), (0, kp - fpk)))
    f1a = fc1_W[:, :_H]
    f1b = jnp.pad(fc1_W[:, _H:], ((0, 0), (0, kp + _H - fc1_W.shape[1])))
    gamma = (bn_gamma / jnp.sqrt(1.0 + 1e-5)).reshape(1, _H)
    beta = bn_beta.reshape(1, _H)

    out = _tc_call(
        _tail_body, jax.ShapeDtypeStruct((1, _G), f32),
        aggr, p, upd_W[_NLAYER - 1], ub[_NLAYER - 1],
        batch.reshape(1, _N), fp_pad,
        gate_W1, gate_b1.reshape(1, _H), gate_W2, gate_b2.reshape(1, 1),
        f1a, f1b, fc1_b.reshape(1, _H), gamma, beta,
        fc2_W, fc2_b.reshape(1, 1))
    return jnp.squeeze(out, axis=0)


# speedup vs baseline: 4.7506x; 1.8205x over previous
import functools

import jax
import jax.numpy as jnp
from jax import lax
from jax.experimental import pallas as pl
from jax.experimental.pallas import tpu as pltpu
from jax.experimental.pallas import tpu_sc as plsc

_N = 10000
_E = 320000
_H = 128
_G = 64
_NLAYER = 4

_NC = 2
_NS = 16
_LANES = 16
_NW = _NC * _NS
_CH = 64
_LSTRIP = 16
_NSTRIP = 10
_NCHUNK = _NSTRIP * _LSTRIP
_EPW = _NCHUNK * _CH
_EPAD = _EPW * _NW
_NTBL = 10240
_DUMMY = _N


def _dot_t(a, w):
    return lax.dot_general(a, w, (((1,), (1,)), ((), ())),
                           precision=lax.Precision.HIGHEST,
                           preferred_element_type=jnp.float32)


def _sc_edge_pass(p, src, dst, ea, we):
    mesh = plsc.VectorSubcoreMesh(core_axis_name="c", subcore_axis_name="s")

    @functools.partial(
        pl.kernel,
        mesh=mesh,
        out_type=jax.ShapeDtypeStruct((_NC, _NTBL, _H), jnp.float32),
        scratch_types=[
            pltpu.VMEM((2 * _LSTRIP, _CH), jnp.int32),
            pltpu.VMEM((2 * _LSTRIP, _CH), jnp.int32),
            pltpu.VMEM((2 * _LSTRIP, _CH), jnp.float32),
            pltpu.VMEM((2, _CH, _H), jnp.float32),
            pltpu.VMEM((2, _CH, _H), jnp.float32),
            pltpu.VMEM((_H,), jnp.float32),
            pltpu.VMEM_SHARED((_NTBL, _H), jnp.float32),
            pltpu.SemaphoreType.DMA,
            pltpu.SemaphoreType.DMA,
            pltpu.SemaphoreType.DMA,
            pltpu.SemaphoreType.DMA,
            pltpu.SemaphoreType.DMA,
        ],
    )
    def k(p_hbm, src_hbm, dst_hbm, ea_hbm, we_hbm, out_hbm,
          src_v, dst_v, ea_v, rows_v, msg_v, we_v, aggr_sh,
          gsem0, gsem1, ssem0, ssem1, stsem):
        c = lax.axis_index("c")
        s = lax.axis_index("s")
        wid = c * _NS + s
        gsem = [gsem0, gsem1]
        ssem = [ssem0, ssem1]

        def strip_copies(st, slot):
            rows = pl.ds(slot * _LSTRIP, _LSTRIP)
            return [
                pltpu.make_async_copy(src_hbm.at[wid, st], src_v.at[rows],
                                      stsem),
                pltpu.make_async_copy(dst_hbm.at[wid, st], dst_v.at[rows],
                                      stsem),
                pltpu.make_async_copy(ea_hbm.at[wid, st], ea_v.at[rows],
                                      stsem),
            ]

        def zrow(r, carry):
            for t in range(_H // _LANES):
                msg_v[0, r, pl.ds(t * _LANES, _LANES)] = jnp.zeros(
                    (_LANES,), jnp.float32)
            return carry
        lax.fori_loop(0, _CH, zrow, 0)
        zpt = _NTBL // _NS
        for b in range(zpt // _CH):
            pltpu.sync_copy(msg_v.at[0],
                            aggr_sh.at[pl.ds(s * zpt + b * _CH, _CH)])
        rem = zpt - (zpt // _CH) * _CH
        if rem:
            pltpu.sync_copy(msg_v.at[0, pl.ds(0, rem)],
                            aggr_sh.at[pl.ds(s * zpt + zpt - rem, rem)])

        for cp in strip_copies(0, 0):
            cp.start()
        pltpu.sync_copy(we_hbm, we_v)
        wreg = [we_v[pl.ds(t * _LANES, _LANES)] for t in range(_H // _LANES)]
        for cp in strip_copies(0, 0):
            cp.wait()
        plsc.subcore_barrier()

        for b in range(2):
            pltpu.async_copy(p_hbm.at[src_v.at[b]], rows_v.at[b], gsem[b])

        npair = _NCHUNK // 2

        def pair_body(t, carry):
            for b in range(2):
                g = t * 2 + b
                st = g // _LSTRIP
                row = g % _LSTRIP
                slot = st % 2
                srow = slot * _LSTRIP + row
                pltpu.make_async_copy(p_hbm.at[src_v.at[srow]],
                                      rows_v.at[b], gsem[b]).wait()

                @pl.when(t > 0)
                def _():
                    pltpu.make_async_copy(msg_v.at[b],
                                          aggr_sh.at[dst_v.at[srow]],
                                          ssem[b]).wait()

                def grp_body(gr, gcarry):
                    eav = ea_v[srow, pl.ds(gr * _LANES, _LANES)]
                    for j in range(_LANES):
                        e = gr * _LANES + j
                        eab = eav[j]
                        for tt in range(_H // _LANES):
                            sl = rows_v[b, e, pl.ds(tt * _LANES, _LANES)]
                            msg_v[b, e, pl.ds(tt * _LANES, _LANES)] = (
                                jnp.maximum(sl + eab * wreg[tt], 0.0))
                    return gcarry
                lax.fori_loop(0, _CH // _LANES, grp_body, 0)

                @pl.when(jnp.logical_and(row == 4, st < _NSTRIP - 1))
                def _():
                    for cp in strip_copies(st + 1, 1 - slot):
                        cp.start()

                @pl.when(jnp.logical_and(row == _LSTRIP - 2,
                                         st < _NSTRIP - 1))
                def _():
                    for cp in strip_copies(st + 1, 1 - slot):
                        cp.wait()

                @pl.when(t < npair - 1)
                def _():
                    g2 = g + 2
                    st2 = g2 // _LSTRIP
                    pltpu.async_copy(
                        p_hbm.at[src_v.at[(st2 % 2) * _LSTRIP
                                          + g2 % _LSTRIP]],
                        rows_v.at[b], gsem[b])

                pltpu.async_copy(msg_v.at[b], aggr_sh.at[dst_v.at[srow]],
                                 ssem[b], add=True)
            return carry
        lax.fori_loop(0, npair, pair_body, 0)

        for b in range(2):
            pltpu.make_async_copy(msg_v.at[b], aggr_sh.at[dst_v.at[0]],
                                  ssem[b]).wait()

        plsc.subcore_barrier()
        rpt = _NTBL // _NS
        pltpu.sync_copy(aggr_sh.at[pl.ds(s * rpt, rpt)],
                        out_hbm.at[c, pl.ds(s * rpt, rpt)])

    return k(p, src, dst, ea, we)


def _p0_body(x_ref, w_ref, b_ref, o_ref):
    o_ref[...] = _dot_t(x_ref[...], w_ref[...]) + b_ref[...]


def _mid_body(a_ref, p_ref, uw_ref, ub_ref, nw_ref, nb_ref, pn_ref):
    a = a_ref[0, :_N] + a_ref[1, :_N] + jnp.maximum(p_ref[...], 0.0)
    h = jnp.maximum(_dot_t(a, uw_ref[...]) + ub_ref[...], 0.0)
    pn_ref[...] = _dot_t(h, nw_ref[...]) + nb_ref[...]


def _tail_body(a_ref, p_ref, uw_ref, ub_ref, batch_ref, fp_ref,
               gw1_ref, gb1_ref, gw2_ref, gb2_ref,
               f1a_ref, f1b_ref, f1b_bias_ref, gamma_ref, beta_ref,
               f2w_ref, f2b_ref, o_ref):
    a = a_ref[0, :_N] + a_ref[1, :_N] + jnp.maximum(p_ref[...], 0.0)
    h = jnp.maximum(_dot_t(a, uw_ref[...]) + ub_ref[...], 0.0)
    g1 = jnp.tanh(_dot_t(h, gw1_ref[...]) + gb1_ref[...])
    gate = jax.nn.sigmoid(
        lax.dot_general(gw2_ref[...], g1, (((1,), (1,)), ((), ())),
                        precision=lax.Precision.HIGHEST,
                        preferred_element_type=jnp.float32)
        + gb2_ref[0, 0])
    seg = lax.broadcasted_iota(jnp.int32, (_G, _N), 0)
    m = (batch_ref[...] == seg).astype(jnp.float32) * gate
    pooled = jnp.dot(m, h, precision=lax.Precision.HIGHEST,
                     preferred_element_type=jnp.float32)
    z = (_dot_t(pooled, f1a_ref[...]) + _dot_t(fp_ref[...], f1b_ref[...])
         + f1b_bias_ref[...])
    z = z * gamma_ref[...] + beta_ref[...]
    z = jnp.maximum(z, 0.0)
    o_ref[...] = lax.dot_general(
        f2w_ref[...], z, (((1,), (1,)), ((), ())),
        precision=lax.Precision.HIGHEST,
        preferred_element_type=jnp.float32) + f2b_ref[0, 0]


def _tc_call(body, out_shape, *args):
    return pl.pallas_call(body, out_shape=out_shape)(*args)


def kernel(x, edge_index, edge_attr, batch, fingerprint,
           lin_W_0, lin_b_0, upd_W_0, upd_b_0,
           lin_W_1, lin_b_1, upd_W_1, upd_b_1,
           lin_W_2, lin_b_2, upd_W_2, upd_b_2,
           lin_W_3, lin_b_3, upd_W_3, upd_b_3,
           gate_W1, gate_b1, gate_W2, gate_b2,
           fc1_W, fc1_b, bn_gamma, bn_beta, fc2_W, fc2_b):
    lin_W = [lin_W_0, lin_W_1, lin_W_2, lin_W_3]
    lin_b = [lin_b_0, lin_b_1, lin_b_2, lin_b_3]
    upd_W = [upd_W_0, upd_W_1, upd_W_2, upd_W_3]
    upd_b = [upd_b_0, upd_b_1, upd_b_2, upd_b_3]

    wh = [w[:, :_H] for w in lin_W]
    we = [w[:, _H] for w in lin_W]
    lb = [b.reshape(1, _H) for b in lin_b]
    ub = [b.reshape(1, _H) for b in upd_b]

    pad = _EPAD - _E
    src_p = jnp.concatenate([edge_index[0],
                             jnp.zeros((pad,), jnp.int32)]
                            ).reshape(_NW, _NSTRIP, _LSTRIP, _CH)
    dst_p = jnp.concatenate([edge_index[1],
                             jnp.full((pad,), _DUMMY, jnp.int32)]
                            ).reshape(_NW, _NSTRIP, _LSTRIP, _CH)
    ea_p = jnp.concatenate([edge_attr[:, 0],
                            jnp.zeros((pad,), jnp.float32)]
                           ).reshape(_NW, _NSTRIP, _LSTRIP, _CH)

    f32 = jnp.float32
    p = _tc_call(_p0_body, jax.ShapeDtypeStruct((_N, _H), f32),
                 x, wh[0], lb[0])

    for l in range(_NLAYER - 1):
        aggr = _sc_edge_pass(p, src_p, dst_p, ea_p, we[l])
        p = _tc_call(_mid_body, jax.ShapeDtypeStruct((_N, _H), f32),
                     aggr, p, upd_W[l], ub[l], wh[l + 1], lb[l + 1])

    aggr = _sc_edge_pass(p, src_p, dst_p, ea_p, we[_NLAYER - 1])

    fpk = fingerprint.shape[1]
    kp = -(-fpk // _H) * _H
    fp_pad = jnp.pad(fingerprint, ((0, 0), (0, kp - fpk)))
    f1a = fc1_W[:, :_H]
    f1b = jnp.pad(fc1_W[:, _H:], ((0, 0), (0, kp + _H - fc1_W.shape[1])))
    gamma = (bn_gamma / jnp.sqrt(1.0 + 1e-5)).reshape(1, _H)
    beta = bn_beta.reshape(1, _H)

    out = _tc_call(
        _tail_body, jax.ShapeDtypeStruct((1, _G), f32),
        aggr, p, upd_W[_NLAYER - 1], ub[_NLAYER - 1],
        batch.reshape(1, _N), fp_pad,
        gate_W1, gate_b1.reshape(1, _H), gate_W2, gate_b2.reshape(1, 1),
        f1a, f1b, fc1_b.reshape(1, _H), gamma, beta,
        fc2_W, fc2_b.reshape(1, 1))
    return jnp.squeeze(out, axis=0)
